# baseline (speedup 1.0000x reference)
import functools

import jax
import jax.numpy as jnp
from jax import lax
from jax.experimental import pallas as pl
from jax.experimental.pallas import tpu as pltpu
from jax.experimental.pallas import tpu_sc as plsc

B = 16384
EMB = 64
HALF = 32
FEAT = 128
NC, NS = 2, 16
NW = NC * NS
BPW = B // NW


def _sc_gather_body(uidx_h, iidx_h, cidx_h, sidx_h,
                    utab_h, itab_h, ctab_h, stab_h,
                    ue_h, ie_h, ce_h, se_h,
                    uidx_v, iidx_v, cidx_v, sidx_v,
                    urows, irows, crows, srows,
                    sem_u, sem_i, sem_c, sem_s):
    wid = lax.axis_index("s") * NC + lax.axis_index("c")
    base = wid * BPW
    sl = pl.ds(base, BPW)
    pltpu.sync_copy(uidx_h.at[sl], uidx_v)
    pltpu.sync_copy(iidx_h.at[sl], iidx_v)
    pltpu.sync_copy(cidx_h.at[sl], cidx_v)
    pltpu.sync_copy(sidx_h.at[sl], sidx_v)
    cu = pltpu.async_copy(utab_h.at[uidx_v], urows, sem_u)
    ci = pltpu.async_copy(itab_h.at[iidx_v], irows, sem_i)
    cc = pltpu.async_copy(ctab_h.at[cidx_v], crows, sem_c)
    cs = pltpu.async_copy(stab_h.at[sidx_v], srows, sem_s)
    cu.wait()
    pltpu.sync_copy(urows, ue_h.at[sl])
    ci.wait()
    pltpu.sync_copy(irows, ie_h.at[sl])
    cc.wait()
    pltpu.sync_copy(crows, ce_h.at[sl])
    cs.wait()
    pltpu.sync_copy(srows, se_h.at[sl])


def _sc_gather(uidx, iidx, cidx, sidx, utab, itab, ctab, stab):
    mesh = plsc.VectorSubcoreMesh(core_axis_name="c", subcore_axis_name="s")
    k = pl.kernel(
        _sc_gather_body,
        out_type=[
            jax.ShapeDtypeStruct((B, EMB), jnp.float32),
            jax.ShapeDtypeStruct((B, EMB), jnp.float32),
            jax.ShapeDtypeStruct((B, HALF), jnp.float32),
            jax.ShapeDtypeStruct((B, HALF), jnp.float32),
        ],
        mesh=mesh,
        scratch_types=[
            pltpu.VMEM((BPW,), jnp.int32),
            pltpu.VMEM((BPW,), jnp.int32),
            pltpu.VMEM((BPW,), jnp.int32),
            pltpu.VMEM((BPW,), jnp.int32),
            pltpu.VMEM((BPW, EMB), jnp.float32),
            pltpu.VMEM((BPW, EMB), jnp.float32),
            pltpu.VMEM((BPW, HALF), jnp.float32),
            pltpu.VMEM((BPW, HALF), jnp.float32),
            pltpu.SemaphoreType.DMA,
            pltpu.SemaphoreType.DMA,
            pltpu.SemaphoreType.DMA,
            pltpu.SemaphoreType.DMA,
        ],
        compiler_params=pltpu.CompilerParams(use_tc_tiling_on_sc=False),
    )
    return k(uidx, iidx, cidx, sidx, utab, itab, ctab, stab)


BM = 2048


def _tc_combine_body(uf_r, wu_r, bu_r, if_r, wi_r, bi_r,
                     ue_r, ce_r, se_r, ie_r, uo_r, io_r):
    up = jnp.dot(uf_r[...], wu_r[...],
                 preferred_element_type=jnp.float32) + bu_r[...]
    ip = jnp.dot(if_r[...], wi_r[...],
                 preferred_element_type=jnp.float32) + bi_r[...]
    uo_r[...] = jnp.concatenate([ue_r[...], ce_r[...], se_r[...], up], axis=-1)
    io_r[...] = jnp.concatenate([ie_r[...], ip], axis=-1)


def _tc_combine(uf, wu, bu, itf, wi, bi, ue, ce, se, ie):
    return pl.pallas_call(
        _tc_combine_body,
        grid=(B // BM,),
        in_specs=[
            pl.BlockSpec((BM, FEAT), lambda i: (i, 0)),
            pl.BlockSpec((FEAT, EMB), lambda i: (0, 0)),
            pl.BlockSpec((1, EMB), lambda i: (0, 0)),
            pl.BlockSpec((BM, FEAT), lambda i: (i, 0)),
            pl.BlockSpec((FEAT, EMB), lambda i: (0, 0)),
            pl.BlockSpec((1, EMB), lambda i: (0, 0)),
            pl.BlockSpec((BM, EMB), lambda i: (i, 0)),
            pl.BlockSpec((BM, HALF), lambda i: (i, 0)),
            pl.BlockSpec((BM, HALF), lambda i: (i, 0)),
            pl.BlockSpec((BM, EMB), lambda i: (i, 0)),
        ],
        out_specs=[
            pl.BlockSpec((BM, 3 * EMB), lambda i: (i, 0)),
            pl.BlockSpec((BM, 2 * EMB), lambda i: (i, 0)),
        ],
        out_shape=[
            jax.ShapeDtypeStruct((B, 3 * EMB), jnp.float32),
            jax.ShapeDtypeStruct((B, 2 * EMB), jnp.float32),
        ],
    )(uf, wu, bu, itf, wi, bi, ue, ce, se, ie)


def kernel(user_idx, user_features, user_color_idx, user_size_idx,
           item_idx, item_features, user_table, item_table,
           color_table, size_table, W_user, b_user, W_item, b_item):
    ue, ie, ce, se = _sc_gather(user_idx, item_idx, user_color_idx,
                                user_size_idx, user_table, item_table,
                                color_table, size_table)
    uo, io = _tc_combine(user_features, W_user, b_user.reshape(1, EMB),
                         item_features, W_item, b_item.reshape(1, EMB),
                         ue, ce, se, ie)
    return uo, io


# speedup vs baseline: 2.3889x; 2.3889x over previous
import jax
import jax.numpy as jnp
from jax import lax
from jax.experimental import pallas as pl
from jax.experimental.pallas import tpu as pltpu
from jax.experimental.pallas import tpu_sc as plsc

B = 16384
EMB = 64
HALF = 32
FEAT = 128
NC, NS = 2, 16
NW = NC * NS
BPW = B // NW
CH = 64
NCHUNK = BPW // CH
NROWTILES = 1000000 // 8


HB = BPW // 2


def _sc_gather_body(uidx_h, iidx_h, utab_h, itab_h,
                    ue_h, ie_h,
                    idx_v, rows, sem):
    wid = lax.axis_index("s") * NC + lax.axis_index("c")
    base = wid * BPW
    lanes = lax.iota(jnp.int32, 16)
    zeros = jnp.zeros((16,), jnp.int32)
    for tab_h, idx_h, emb_h in ((utab_h, uidx_h, ue_h), (itab_h, iidx_h, ie_h)):
        pltpu.sync_copy(idx_h.at[pl.ds(base, BPW)], idx_v)
        for h in range(2):
            def blk_body(k, _):
                raw = idx_v[pl.ds(h * HB + k * 16, 16)]
                for l in range(16):
                    ix = jnp.max(jnp.where(lanes == l, raw, zeros), axis=0)
                    pltpu.async_copy(
                        tab_h.at[lax.shift_right_logical(ix, 3),
                                 lax.bitwise_and(ix, 7)],
                        rows.at[k * 16 + l], sem)
                return _
            lax.fori_loop(0, HB // 16, blk_body, 0)
            hsl = pl.ds(base + h * HB, HB)
            pltpu.make_async_copy(emb_h.at[hsl], rows, sem).wait()
            pltpu.sync_copy(rows, emb_h.at[hsl])


def _sc_gather(uidx, iidx, utab, itab):
    mesh = plsc.VectorSubcoreMesh(core_axis_name="c", subcore_axis_name="s")
    k = pl.kernel(
        _sc_gather_body,
        out_type=[
            jax.ShapeDtypeStruct((B, EMB), jnp.float32),
            jax.ShapeDtypeStruct((B, EMB), jnp.float32),
        ],
        mesh=mesh,
        scratch_types=[
            pltpu.VMEM((BPW,), jnp.int32),
            pltpu.VMEM((HB, EMB), jnp.float32),
            pltpu.SemaphoreType.DMA,
        ],
        compiler_params=pltpu.CompilerParams(use_tc_tiling_on_sc=True,
                                             needs_layout_passes=False),
    )
    return k(uidx, iidx, utab.reshape(NROWTILES, 8, EMB),
             itab.reshape(NROWTILES, 8, EMB))


BM = 2048


def _tc_combine_body(uf_r, wu_r, bu_r, if_r, wi_r, bi_r,
                     ue_r, ie_r, ct_r, st_r, ci_r, si_r, uo_r, io_r):
    up = jnp.dot(uf_r[...], wu_r[...],
                 preferred_element_type=jnp.float32) + bu_r[...]
    ip = jnp.dot(if_r[...], wi_r[...],
                 preferred_element_type=jnp.float32) + bi_r[...]
    conehot = (ci_r[...] == lax.broadcasted_iota(jnp.int32, (1, 22), 1)
               ).astype(jnp.float32)
    sonehot = (si_r[...] == lax.broadcasted_iota(jnp.int32, (1, 18), 1)
               ).astype(jnp.float32)
    ce = jnp.dot(conehot, ct_r[...], preferred_element_type=jnp.float32)
    se = jnp.dot(sonehot, st_r[...], preferred_element_type=jnp.float32)
    uo_r[...] = jnp.concatenate([ue_r[...], ce, se, up], axis=-1)
    io_r[...] = jnp.concatenate([ie_r[...], ip], axis=-1)


def _tc_combine(uf, wu, bu, itf, wi, bi, ue, ie, ctab, stab, cidx, sidx):
    return pl.pallas_call(
        _tc_combine_body,
        grid=(B // BM,),
        in_specs=[
            pl.BlockSpec((BM, FEAT), lambda i: (i, 0)),
            pl.BlockSpec((FEAT, EMB), lambda i: (0, 0)),
            pl.BlockSpec((1, EMB), lambda i: (0, 0)),
            pl.BlockSpec((BM, FEAT), lambda i: (i, 0)),
            pl.BlockSpec((FEAT, EMB), lambda i: (0, 0)),
            pl.BlockSpec((1, EMB), lambda i: (0, 0)),
            pl.BlockSpec((BM, EMB), lambda i: (i, 0)),
            pl.BlockSpec((BM, EMB), lambda i: (i, 0)),
            pl.BlockSpec((22, HALF), lambda i: (0, 0)),
            pl.BlockSpec((18, HALF), lambda i: (0, 0)),
            pl.BlockSpec((BM, 1), lambda i: (i, 0)),
            pl.BlockSpec((BM, 1), lambda i: (i, 0)),
        ],
        out_specs=[
            pl.BlockSpec((BM, 3 * EMB), lambda i: (i, 0)),
            pl.BlockSpec((BM, 2 * EMB), lambda i: (i, 0)),
        ],
        out_shape=[
            jax.ShapeDtypeStruct((B, 3 * EMB), jnp.float32),
            jax.ShapeDtypeStruct((B, 2 * EMB), jnp.float32),
        ],
    )(uf, wu, bu, itf, wi, bi, ue, ie, ctab, stab, cidx, sidx)


def kernel(user_idx, user_features, user_color_idx, user_size_idx,
           item_idx, item_features, user_table, item_table,
           color_table, size_table, W_user, b_user, W_item, b_item):
    ue, ie = _sc_gather(user_idx, item_idx, user_table, item_table)
    uo, io = _tc_combine(user_features, W_user, b_user.reshape(1, EMB),
                         item_features, W_item, b_item.reshape(1, EMB),
                         ue, ie, color_table, size_table,
                         user_color_idx.reshape(B, 1),
                         user_size_idx.reshape(B, 1))
    return uo, io
